# Initial kernel scaffold; baseline (speedup 1.0000x reference)
import jax
import jax.numpy as jnp
from jax.experimental import pallas as pl


def kernel(xyz, features, edges, W1, b1, g1, be1, W2, b2, g2, be2):
    raise NotImplementedError("write your pallas kernel here")


# speedup vs baseline: 942.9276x; 942.9276x over previous
import functools

import jax
import jax.numpy as jnp
from jax.experimental import pallas as pl
from jax.experimental.pallas import tpu as pltpu

N_BLK = 1000
E_BLK = 640


def _node_prep_kernel(ivf_ref, xyzp_ref, w1a_ref, w1b_ref, b1_ref, a_ref, q_ref):
    q = jnp.dot(xyzp_ref[...], w1b_ref[...], preferred_element_type=jnp.float32)
    a = jnp.dot(ivf_ref[...], w1a_ref[...], preferred_element_type=jnp.float32)
    a_ref[...] = a + q + b1_ref[...]
    q_ref[...] = q


def _node_prep(ivf, xyzp, w1aT, w1bpT, b1):
    n = ivf.shape[0]
    c = ivf.shape[1]
    grid = (n // N_BLK,)
    return pl.pallas_call(
        _node_prep_kernel,
        grid=grid,
        in_specs=[
            pl.BlockSpec((N_BLK, c), lambda i: (i, 0)),
            pl.BlockSpec((N_BLK, 128), lambda i: (i, 0)),
            pl.BlockSpec((c, c), lambda i: (0, 0)),
            pl.BlockSpec((128, c), lambda i: (0, 0)),
            pl.BlockSpec((1, c), lambda i: (0, 0)),
        ],
        out_specs=[
            pl.BlockSpec((N_BLK, c), lambda i: (i, 0)),
            pl.BlockSpec((N_BLK, c), lambda i: (i, 0)),
        ],
        out_shape=[
            jax.ShapeDtypeStruct((n, c), jnp.float32),
            jax.ShapeDtypeStruct((n, c), jnp.float32),
        ],
    )(ivf, xyzp, w1aT, w1bpT, b1)


def _edge_ln_kernel(asrc_ref, qd_ref, g_ref, be_ref, h_ref):
    x = jnp.maximum(asrc_ref[...] - qd_ref[...], 0.0)
    mu = jnp.mean(x, axis=-1, keepdims=True)
    xc = x - mu
    var = jnp.mean(xc * xc, axis=-1, keepdims=True)
    h_ref[...] = xc * jax.lax.rsqrt(var + 1e-5) * g_ref[...] + be_ref[...]


def _edge_ln(asrc, qd, g1, be1):
    e, c = asrc.shape
    return pl.pallas_call(
        _edge_ln_kernel,
        grid=(e // E_BLK,),
        in_specs=[
            pl.BlockSpec((E_BLK, c), lambda i: (i, 0)),
            pl.BlockSpec((E_BLK, c), lambda i: (i, 0)),
            pl.BlockSpec((1, c), lambda i: (0, 0)),
            pl.BlockSpec((1, c), lambda i: (0, 0)),
        ],
        out_specs=pl.BlockSpec((E_BLK, c), lambda i: (i, 0)),
        out_shape=jax.ShapeDtypeStruct((e, c), jnp.float32),
    )(asrc, qd, g1, be1)


def _update_kernel(agg_ref, ivf_ref, w2_ref, b2_ref, g_ref, be_ref, out_ref):
    u = jnp.dot(agg_ref[...], w2_ref[...], preferred_element_type=jnp.float32)
    u = jnp.maximum(u + b2_ref[...], 0.0)
    mu = jnp.mean(u, axis=-1, keepdims=True)
    uc = u - mu
    var = jnp.mean(uc * uc, axis=-1, keepdims=True)
    out_ref[...] = uc * jax.lax.rsqrt(var + 1e-5) * g_ref[...] + be_ref[...] + ivf_ref[...]


def _update(agg, ivf, w2T, b2, g2, be2):
    n, c = agg.shape
    return pl.pallas_call(
        _update_kernel,
        grid=(n // N_BLK,),
        in_specs=[
            pl.BlockSpec((N_BLK, c), lambda i: (i, 0)),
            pl.BlockSpec((N_BLK, c), lambda i: (i, 0)),
            pl.BlockSpec((c, c), lambda i: (0, 0)),
            pl.BlockSpec((1, c), lambda i: (0, 0)),
            pl.BlockSpec((1, c), lambda i: (0, 0)),
            pl.BlockSpec((1, c), lambda i: (0, 0)),
        ],
        out_specs=pl.BlockSpec((N_BLK, c), lambda i: (i, 0)),
        out_shape=jax.ShapeDtypeStruct((n, c), jnp.float32),
    )(agg, ivf, w2T, b2, g2, be2)


def kernel(xyz, features, edges, W1, b1, g1, be1, W2, b2, g2, be2):
    n = xyz.shape[1]
    c = features.shape[1]
    ivf = features[0].T
    src = edges[0, :, 0]
    dst = edges[0, :, 1]
    xyzp = jnp.pad(xyz[0], ((0, 0), (0, 125)))
    w1aT = W1[:, :c].T
    w1bpT = jnp.pad(W1[:, c:].T, ((0, 125), (0, 0)))

    A, Q = _node_prep(ivf, xyzp, w1aT, w1bpT, b1[None, :])

    asrc = jnp.take(A, src, axis=0)
    qd = jnp.take(Q, dst, axis=0)
    h = _edge_ln(asrc, qd, g1[None, :], be1[None, :])
    agg = jax.ops.segment_max(h, dst, num_segments=n)
    agg = jnp.where(jnp.isneginf(agg), 0.0, agg)

    u = _update(agg, ivf, W2.T, b2[None, :], g2[None, :], be2[None, :])
    return u.T[None]
